# Initial kernel scaffold; baseline (speedup 1.0000x reference)
import jax
import jax.numpy as jnp
from jax.experimental import pallas as pl


def kernel(x, edge_index, batch, w_in, b_in, gn_weight, gn_bias, gn_mean_scale, ggc_w, gru_w_ih, gru_w_hh, gru_b_ih, gru_b_hh, ln_msg_w, ln_msg_b, ln_edge_w, ln_edge_b, ec_w1, ec_b1, ec_w2, ec_b2, w_proj, b_proj):
    raise NotImplementedError("write your pallas kernel here")


# speedup vs baseline: 1.1361x; 1.1361x over previous
import functools
import math

import jax
import jax.numpy as jnp
from jax import lax
from jax.experimental import pallas as pl

N = 10000
E = 160000
D = 128
H = 256
L = 3
P = 128
G = 64

BN = 1000
BE = 1000
NB = N // BN
EB = E // BE

_SQRT2 = math.sqrt(2.0)


def _gelu(v):
    return 0.5 * v * (1.0 + lax.erf(v / _SQRT2))


def _ln(v, w, b):
    mu = jnp.mean(v, axis=-1, keepdims=True)
    var = jnp.mean((v - mu) ** 2, axis=-1, keepdims=True)
    return (v - mu) * lax.rsqrt(var + 1e-5) * w + b


def _mm(a, b):
    return jnp.dot(a, b, preferred_element_type=jnp.float32)


def _k_inproj(x_ref, w_ref, b_ref, o_ref):
    o_ref[...] = _mm(x_ref[...], w_ref[...]) + b_ref[...]


def in_proj(x, w_in, b_in):
    return pl.pallas_call(
        _k_inproj,
        grid=(NB,),
        in_specs=[
            pl.BlockSpec((BN, D), lambda i: (i, 0)),
            pl.BlockSpec((D, H), lambda i: (0, 0)),
            pl.BlockSpec((1, H), lambda i: (0, 0)),
        ],
        out_specs=pl.BlockSpec((BN, H), lambda i: (i, 0)),
        out_shape=jax.ShapeDtypeStruct((N, H), jnp.float32),
    )(x, w_in, b_in.reshape(1, H))


def _k_stats(x_ref, b_ref, sum_ref, sq_ref, cnt_ref):
    xb = x_ref[...]
    bb = b_ref[0, 0, :]
    onehot = (bb[:, None] == lax.broadcasted_iota(jnp.int32, (BN, G), 1)).astype(jnp.float32)
    ps = _mm(onehot.T, xb)
    pq = _mm(onehot.T, xb * xb)
    pc = jnp.sum(onehot, axis=0, keepdims=True)

    @pl.when(pl.program_id(0) == 0)
    def _():
        sum_ref[...] = jnp.zeros_like(sum_ref)
        sq_ref[...] = jnp.zeros_like(sq_ref)
        cnt_ref[...] = jnp.zeros_like(cnt_ref)

    sum_ref[...] += ps
    sq_ref[...] += pq
    cnt_ref[...] += pc


def graph_stats(x0, batch3):
    return pl.pallas_call(
        _k_stats,
        grid=(NB,),
        in_specs=[
            pl.BlockSpec((BN, H), lambda i: (i, 0)),
            pl.BlockSpec((1, 1, BN), lambda i: (i, 0, 0)),
        ],
        out_specs=[
            pl.BlockSpec((G, H), lambda i: (0, 0)),
            pl.BlockSpec((G, H), lambda i: (0, 0)),
            pl.BlockSpec((1, G), lambda i: (0, 0)),
        ],
        out_shape=[
            jax.ShapeDtypeStruct((G, H), jnp.float32),
            jax.ShapeDtypeStruct((G, H), jnp.float32),
            jax.ShapeDtypeStruct((1, G), jnp.float32),
        ],
    )(x0, batch3)


def _k_gn(x_ref, b_ref, sum_ref, sq_ref, cnt_ref, ms_ref, w_ref, bia_ref, o_ref):
    xb = x_ref[...]
    bb = b_ref[0, 0, :]
    cnt = jnp.maximum(cnt_ref[0, :], 1.0)[:, None]
    mean = sum_ref[...] / cnt
    s = ms_ref[...]
    m2 = mean * mean
    var = sq_ref[...] / cnt - 2.0 * s * m2 + (s * s) * m2
    onehot = (bb[:, None] == lax.broadcasted_iota(jnp.int32, (BN, G), 1)).astype(jnp.float32)
    mean_b = _mm(onehot, mean * s)
    var_b = _mm(onehot, var)
    o_ref[...] = (xb - mean_b) * lax.rsqrt(var_b + 1e-5) * w_ref[...] + bia_ref[...]


def gn_apply(x0, batch3, sums, sq, cnt, gn_mean_scale, gn_weight, gn_bias):
    return pl.pallas_call(
        _k_gn,
        grid=(NB,),
        in_specs=[
            pl.BlockSpec((BN, H), lambda i: (i, 0)),
            pl.BlockSpec((1, 1, BN), lambda i: (i, 0, 0)),
            pl.BlockSpec((G, H), lambda i: (0, 0)),
            pl.BlockSpec((G, H), lambda i: (0, 0)),
            pl.BlockSpec((1, G), lambda i: (0, 0)),
            pl.BlockSpec((1, H), lambda i: (0, 0)),
            pl.BlockSpec((1, H), lambda i: (0, 0)),
            pl.BlockSpec((1, H), lambda i: (0, 0)),
        ],
        out_specs=pl.BlockSpec((BN, H), lambda i: (i, 0)),
        out_shape=jax.ShapeDtypeStruct((N, H), jnp.float32),
    )(x0, batch3, sums, sq, cnt, gn_mean_scale.reshape(1, H),
      gn_weight.reshape(1, H), gn_bias.reshape(1, H))


def _k_mm(a_ref, w_ref, o_ref):
    o_ref[...] = _mm(a_ref[...], w_ref[...])


def node_mm(h, w):
    return pl.pallas_call(
        _k_mm,
        grid=(NB,),
        in_specs=[
            pl.BlockSpec((BN, H), lambda i: (i, 0)),
            pl.BlockSpec((H, H), lambda i: (0, 0)),
        ],
        out_specs=pl.BlockSpec((BN, H), lambda i: (i, 0)),
        out_shape=jax.ShapeDtypeStruct((N, H), jnp.float32),
    )(h, w)


def _k_gru(h_ref, agg_ref, wih_ref, whh_ref, bih_ref, bhh_ref, o_ref):
    h = h_ref[...]
    gi = _mm(agg_ref[...], wih_ref[...]) + bih_ref[...]
    gh = _mm(h, whh_ref[...]) + bhh_ref[...]
    i_r, i_z, i_n = gi[:, :H], gi[:, H:2 * H], gi[:, 2 * H:]
    h_r, h_z, h_n = gh[:, :H], gh[:, H:2 * H], gh[:, 2 * H:]
    r = jax.nn.sigmoid(i_r + h_r)
    z = jax.nn.sigmoid(i_z + h_z)
    n = jnp.tanh(i_n + r * h_n)
    o_ref[...] = (1.0 - z) * n + z * h


def gru_cell(h, agg, w_ihT, w_hhT, b_ih, b_hh):
    return pl.pallas_call(
        _k_gru,
        grid=(NB,),
        in_specs=[
            pl.BlockSpec((BN, H), lambda i: (i, 0)),
            pl.BlockSpec((BN, H), lambda i: (i, 0)),
            pl.BlockSpec((H, 3 * H), lambda i: (0, 0)),
            pl.BlockSpec((H, 3 * H), lambda i: (0, 0)),
            pl.BlockSpec((1, 3 * H), lambda i: (0, 0)),
            pl.BlockSpec((1, 3 * H), lambda i: (0, 0)),
        ],
        out_specs=pl.BlockSpec((BN, H), lambda i: (i, 0)),
        out_shape=jax.ShapeDtypeStruct((N, H), jnp.float32),
    )(h, agg, w_ihT, w_hhT, b_ih.reshape(1, 3 * H), b_hh.reshape(1, 3 * H))


def _k_msgln(h_ref, x0_ref, w_ref, b_ref, o_ref):
    o_ref[...] = x0_ref[...] + _ln(_gelu(h_ref[...]), w_ref[...], b_ref[...])


def msg_ln(h, x0n, ln_msg_w, ln_msg_b):
    return pl.pallas_call(
        _k_msgln,
        grid=(NB,),
        in_specs=[
            pl.BlockSpec((BN, H), lambda i: (i, 0)),
            pl.BlockSpec((BN, H), lambda i: (i, 0)),
            pl.BlockSpec((1, H), lambda i: (0, 0)),
            pl.BlockSpec((1, H), lambda i: (0, 0)),
        ],
        out_specs=pl.BlockSpec((BN, H), lambda i: (i, 0)),
        out_shape=jax.ShapeDtypeStruct((N, H), jnp.float32),
    )(h, x0n, ln_msg_w.reshape(1, H), ln_msg_b.reshape(1, H))


def _k_ec(xi_ref, xj_ref, wa_ref, wb_ref, b1_ref, w2_ref, b2_ref, o_ref):
    t = _mm(xi_ref[...], wa_ref[...]) + _mm(xj_ref[...], wb_ref[...]) + b1_ref[...]
    o_ref[...] = _mm(_gelu(t), w2_ref[...]) + b2_ref[...]


def ec_mlp(xi, xj, w_a, w_b, ec_b1, ec_w2, ec_b2):
    return pl.pallas_call(
        _k_ec,
        grid=(EB,),
        in_specs=[
            pl.BlockSpec((BE, H), lambda i: (i, 0)),
            pl.BlockSpec((BE, H), lambda i: (i, 0)),
            pl.BlockSpec((H, H), lambda i: (0, 0)),
            pl.BlockSpec((H, H), lambda i: (0, 0)),
            pl.BlockSpec((1, H), lambda i: (0, 0)),
            pl.BlockSpec((H, H), lambda i: (0, 0)),
            pl.BlockSpec((1, H), lambda i: (0, 0)),
        ],
        out_specs=pl.BlockSpec((BE, H), lambda i: (i, 0)),
        out_shape=jax.ShapeDtypeStruct((E, H), jnp.float32),
    )(xi, xj, w_a, w_b, ec_b1.reshape(1, H), ec_w2, ec_b2.reshape(1, H))


def _k_fin(xn_ref, ec_ref, b_ref, w_ref, bia_ref, pool_ref):
    ec = ec_ref[...]
    ec = jnp.where(jnp.isfinite(ec), ec, 0.0)
    xn2 = xn_ref[...] + _ln(_gelu(ec), w_ref[...], bia_ref[...])
    bb = b_ref[0, 0, :]
    onehot = (bb[:, None] == lax.broadcasted_iota(jnp.int32, (BN, G), 1)).astype(jnp.float32)
    ps = _mm(onehot.T, xn2)

    @pl.when(pl.program_id(0) == 0)
    def _():
        pool_ref[...] = jnp.zeros_like(pool_ref)

    pool_ref[...] += ps


def fin_pool(xn, ecm, batch3, ln_edge_w, ln_edge_b):
    return pl.pallas_call(
        _k_fin,
        grid=(NB,),
        in_specs=[
            pl.BlockSpec((BN, H), lambda i: (i, 0)),
            pl.BlockSpec((BN, H), lambda i: (i, 0)),
            pl.BlockSpec((1, 1, BN), lambda i: (i, 0, 0)),
            pl.BlockSpec((1, H), lambda i: (0, 0)),
            pl.BlockSpec((1, H), lambda i: (0, 0)),
        ],
        out_specs=pl.BlockSpec((G, H), lambda i: (0, 0)),
        out_shape=jax.ShapeDtypeStruct((G, H), jnp.float32),
    )(xn, ecm, batch3, ln_edge_w.reshape(1, H), ln_edge_b.reshape(1, H))


def _k_proj(s_ref, c_ref, w_ref, b_ref, o_ref):
    cnt = jnp.maximum(c_ref[0, :], 1.0)[:, None]
    o_ref[...] = _mm(s_ref[...] / cnt, w_ref[...]) + b_ref[...]


def pool_proj(pool_sums, cnt, w_proj, b_proj):
    return pl.pallas_call(
        _k_proj,
        grid=(1,),
        in_specs=[
            pl.BlockSpec((G, H), lambda i: (0, 0)),
            pl.BlockSpec((1, G), lambda i: (0, 0)),
            pl.BlockSpec((H, P), lambda i: (0, 0)),
            pl.BlockSpec((1, P), lambda i: (0, 0)),
        ],
        out_specs=pl.BlockSpec((G, P), lambda i: (0, 0)),
        out_shape=jax.ShapeDtypeStruct((G, P), jnp.float32),
    )(pool_sums, cnt, w_proj, b_proj.reshape(1, P))


def kernel(x, edge_index, batch, w_in, b_in, gn_weight, gn_bias, gn_mean_scale,
           ggc_w, gru_w_ih, gru_w_hh, gru_b_ih, gru_b_hh,
           ln_msg_w, ln_msg_b, ln_edge_w, ln_edge_b,
           ec_w1, ec_b1, ec_w2, ec_b2, w_proj, b_proj):
    src = edge_index[0]
    dst = edge_index[1]
    batch3 = batch.reshape(NB, 1, BN)

    x0 = in_proj(x, w_in, b_in)
    sums, sq, cnt = graph_stats(x0, batch3)
    x0n = gn_apply(x0, batch3, sums, sq, cnt, gn_mean_scale, gn_weight, gn_bias)

    w_ihT = gru_w_ih.T
    w_hhT = gru_w_hh.T
    h = x0n
    for i in range(L):
        m = node_mm(h, ggc_w[i])
        agg = jax.ops.segment_sum(m[src], dst, num_segments=N)
        h = gru_cell(h, agg, w_ihT, w_hhT, gru_b_ih, gru_b_hh)

    xn = msg_ln(h, x0n, ln_msg_w, ln_msg_b)

    w_a = ec_w1[:H] - ec_w1[H:]
    w_b = ec_w1[H:]
    xi = xn[dst]
    xj = xn[src]
    me = ec_mlp(xi, xj, w_a, w_b, ec_b1, ec_w2, ec_b2)
    ecm = jax.ops.segment_max(me, dst, num_segments=N)

    pool_sums = fin_pool(xn, ecm, batch3, ln_edge_w, ln_edge_b)
    return pool_proj(pool_sums, cnt, w_proj, b_proj)


# speedup vs baseline: 2.3225x; 2.0442x over previous
import functools
import math

import jax
import jax.numpy as jnp
from jax import lax
from jax.experimental import pallas as pl
from jax.experimental.pallas import tpu as pltpu
from jax.experimental.pallas import tpu_sc as plsc

N = 10000
E = 160000
D = 128
H = 256
L = 3
P = 128
G = 64

BN = 1000
BE = 1000
NB = N // BN
EB = E // BE

HH = H // 2
NSUB = 16
CE = 200
EPS = E // NSUB
NCH = EPS // CE
NPAD = 10240
RPS = NPAD // NSUB
RLAST = N - 15 * RPS

_SQRT2 = math.sqrt(2.0)


def _gelu(v):
    return 0.5 * v * (1.0 + lax.erf(v / _SQRT2))


def _ln(v, w, b):
    mu = jnp.mean(v, axis=-1, keepdims=True)
    var = jnp.mean((v - mu) ** 2, axis=-1, keepdims=True)
    return (v - mu) * lax.rsqrt(var + 1e-5) * w + b


def _mm(a, b):
    return jnp.dot(a, b, preferred_element_type=jnp.float32)


def _k_inproj(x_ref, w_ref, b_ref, o_ref):
    o_ref[...] = _mm(x_ref[...], w_ref[...]) + b_ref[...]


def in_proj(x, w_in, b_in):
    return pl.pallas_call(
        _k_inproj,
        grid=(NB,),
        in_specs=[
            pl.BlockSpec((BN, D), lambda i: (i, 0)),
            pl.BlockSpec((D, H), lambda i: (0, 0)),
            pl.BlockSpec((1, H), lambda i: (0, 0)),
        ],
        out_specs=pl.BlockSpec((BN, H), lambda i: (i, 0)),
        out_shape=jax.ShapeDtypeStruct((N, H), jnp.float32),
    )(x, w_in, b_in.reshape(1, H))


def _k_stats(x_ref, b_ref, sum_ref, sq_ref, cnt_ref):
    xb = x_ref[...]
    bb = b_ref[0, 0, :]
    onehot = (bb[:, None] == lax.broadcasted_iota(jnp.int32, (BN, G), 1)).astype(jnp.float32)
    ps = _mm(onehot.T, xb)
    pq = _mm(onehot.T, xb * xb)
    pc = jnp.sum(onehot, axis=0, keepdims=True)

    @pl.when(pl.program_id(0) == 0)
    def _():
        sum_ref[...] = jnp.zeros_like(sum_ref)
        sq_ref[...] = jnp.zeros_like(sq_ref)
        cnt_ref[...] = jnp.zeros_like(cnt_ref)

    sum_ref[...] += ps
    sq_ref[...] += pq
    cnt_ref[...] += pc


def graph_stats(x0, batch3):
    return pl.pallas_call(
        _k_stats,
        grid=(NB,),
        in_specs=[
            pl.BlockSpec((BN, H), lambda i: (i, 0)),
            pl.BlockSpec((1, 1, BN), lambda i: (i, 0, 0)),
        ],
        out_specs=[
            pl.BlockSpec((G, H), lambda i: (0, 0)),
            pl.BlockSpec((G, H), lambda i: (0, 0)),
            pl.BlockSpec((1, G), lambda i: (0, 0)),
        ],
        out_shape=[
            jax.ShapeDtypeStruct((G, H), jnp.float32),
            jax.ShapeDtypeStruct((G, H), jnp.float32),
            jax.ShapeDtypeStruct((1, G), jnp.float32),
        ],
    )(x0, batch3)


def _k_gn(x_ref, b_ref, sum_ref, sq_ref, cnt_ref, ms_ref, w_ref, bia_ref, o_ref):
    xb = x_ref[...]
    bb = b_ref[0, 0, :]
    cnt = jnp.maximum(cnt_ref[0, :], 1.0)[:, None]
    mean = sum_ref[...] / cnt
    s = ms_ref[...]
    m2 = mean * mean
    var = sq_ref[...] / cnt - 2.0 * s * m2 + (s * s) * m2
    onehot = (bb[:, None] == lax.broadcasted_iota(jnp.int32, (BN, G), 1)).astype(jnp.float32)
    mean_b = _mm(onehot, mean * s)
    var_b = _mm(onehot, var)
    o_ref[...] = (xb - mean_b) * lax.rsqrt(var_b + 1e-5) * w_ref[...] + bia_ref[...]


def gn_apply(x0, batch3, sums, sq, cnt, gn_mean_scale, gn_weight, gn_bias):
    return pl.pallas_call(
        _k_gn,
        grid=(NB,),
        in_specs=[
            pl.BlockSpec((BN, H), lambda i: (i, 0)),
            pl.BlockSpec((1, 1, BN), lambda i: (i, 0, 0)),
            pl.BlockSpec((G, H), lambda i: (0, 0)),
            pl.BlockSpec((G, H), lambda i: (0, 0)),
            pl.BlockSpec((1, G), lambda i: (0, 0)),
            pl.BlockSpec((1, H), lambda i: (0, 0)),
            pl.BlockSpec((1, H), lambda i: (0, 0)),
            pl.BlockSpec((1, H), lambda i: (0, 0)),
        ],
        out_specs=pl.BlockSpec((BN, H), lambda i: (i, 0)),
        out_shape=jax.ShapeDtypeStruct((N, H), jnp.float32),
    )(x0, batch3, sums, sq, cnt, gn_mean_scale.reshape(1, H),
      gn_weight.reshape(1, H), gn_bias.reshape(1, H))


def _k_mm2(a_ref, w_ref, o0_ref, o1_ref):
    r = _mm(a_ref[...], w_ref[...])
    o0_ref[...] = r[:, :HH]
    o1_ref[...] = r[:, HH:]


def node_mm2(h, w):
    return pl.pallas_call(
        _k_mm2,
        grid=(NB,),
        in_specs=[
            pl.BlockSpec((BN, H), lambda i: (i, 0)),
            pl.BlockSpec((H, H), lambda i: (0, 0)),
        ],
        out_specs=[
            pl.BlockSpec((BN, HH), lambda i: (i, 0)),
            pl.BlockSpec((BN, HH), lambda i: (i, 0)),
        ],
        out_shape=[
            jax.ShapeDtypeStruct((N, HH), jnp.float32),
            jax.ShapeDtypeStruct((N, HH), jnp.float32),
        ],
    )(h, w)


def _sc_scatter_body(m0, m1, src, dst, zer, o0, o1, acc, idx_s, idx_d, rows, sem):
    cid = lax.axis_index("c")
    sid = lax.axis_index("s")
    pltpu.sync_copy(zer, acc.at[pl.ds(sid * RPS, RPS)])
    plsc.subcore_barrier()

    def body(i, carry):
        base = sid * EPS + i * CE
        pltpu.sync_copy(src.at[pl.ds(base, CE)], idx_s)
        pltpu.sync_copy(dst.at[pl.ds(base, CE)], idx_d)

        @pl.when(cid == 0)
        def _():
            pltpu.async_copy(m0.at[idx_s], rows, sem).wait()

        @pl.when(cid == 1)
        def _():
            pltpu.async_copy(m1.at[idx_s], rows, sem).wait()

        pltpu.sync_copy(rows, acc.at[idx_d], add=True)
        return carry

    lax.fori_loop(0, NCH, body, 0)
    plsc.subcore_barrier()

    @pl.when(jnp.logical_and(cid == 0, sid < 15))
    def _():
        pltpu.sync_copy(acc.at[pl.ds(sid * RPS, RPS)], o0.at[pl.ds(sid * RPS, RPS)])

    @pl.when(jnp.logical_and(cid == 1, sid < 15))
    def _():
        pltpu.sync_copy(acc.at[pl.ds(sid * RPS, RPS)], o1.at[pl.ds(sid * RPS, RPS)])

    @pl.when(jnp.logical_and(cid == 0, sid == 15))
    def _():
        pltpu.sync_copy(acc.at[pl.ds(15 * RPS, RLAST)], o0.at[pl.ds(15 * RPS, RLAST)])

    @pl.when(jnp.logical_and(cid == 1, sid == 15))
    def _():
        pltpu.sync_copy(acc.at[pl.ds(15 * RPS, RLAST)], o1.at[pl.ds(15 * RPS, RLAST)])


def sc_scatter_add(m0, m1, src, dst, zer):
    mesh = plsc.VectorSubcoreMesh(core_axis_name="c", subcore_axis_name="s")
    f = functools.partial(
        pl.kernel,
        mesh=mesh,
        out_type=[
            jax.ShapeDtypeStruct((N, HH), jnp.float32),
            jax.ShapeDtypeStruct((N, HH), jnp.float32),
        ],
        scratch_types=[
            pltpu.VMEM_SHARED((NPAD, HH), jnp.float32),
            pltpu.VMEM((CE,), jnp.int32),
            pltpu.VMEM((CE,), jnp.int32),
            pltpu.VMEM((CE, HH), jnp.float32),
            pltpu.SemaphoreType.DMA,
        ],
    )(_sc_scatter_body)
    return f(m0, m1, src, dst, zer)


def _k_gru(h_ref, a0_ref, a1_ref, wih_ref, whh_ref, bih_ref, bhh_ref, o_ref):
    h = h_ref[...]
    wih = wih_ref[...]
    gi = _mm(a0_ref[...], wih[:HH, :]) + _mm(a1_ref[...], wih[HH:, :]) + bih_ref[...]
    gh = _mm(h, whh_ref[...]) + bhh_ref[...]
    i_r, i_z, i_n = gi[:, :H], gi[:, H:2 * H], gi[:, 2 * H:]
    h_r, h_z, h_n = gh[:, :H], gh[:, H:2 * H], gh[:, 2 * H:]
    r = jax.nn.sigmoid(i_r + h_r)
    z = jax.nn.sigmoid(i_z + h_z)
    n = jnp.tanh(i_n + r * h_n)
    o_ref[...] = (1.0 - z) * n + z * h


def gru_cell(h, a0, a1, w_ihT, w_hhT, b_ih, b_hh):
    return pl.pallas_call(
        _k_gru,
        grid=(NB,),
        in_specs=[
            pl.BlockSpec((BN, H), lambda i: (i, 0)),
            pl.BlockSpec((BN, HH), lambda i: (i, 0)),
            pl.BlockSpec((BN, HH), lambda i: (i, 0)),
            pl.BlockSpec((H, 3 * H), lambda i: (0, 0)),
            pl.BlockSpec((H, 3 * H), lambda i: (0, 0)),
            pl.BlockSpec((1, 3 * H), lambda i: (0, 0)),
            pl.BlockSpec((1, 3 * H), lambda i: (0, 0)),
        ],
        out_specs=pl.BlockSpec((BN, H), lambda i: (i, 0)),
        out_shape=jax.ShapeDtypeStruct((N, H), jnp.float32),
    )(h, a0, a1, w_ihT, w_hhT, b_ih.reshape(1, 3 * H), b_hh.reshape(1, 3 * H))


def _k_msgln(h_ref, x0_ref, w_ref, b_ref, o_ref):
    o_ref[...] = x0_ref[...] + _ln(_gelu(h_ref[...]), w_ref[...], b_ref[...])


def msg_ln(h, x0n, ln_msg_w, ln_msg_b):
    return pl.pallas_call(
        _k_msgln,
        grid=(NB,),
        in_specs=[
            pl.BlockSpec((BN, H), lambda i: (i, 0)),
            pl.BlockSpec((BN, H), lambda i: (i, 0)),
            pl.BlockSpec((1, H), lambda i: (0, 0)),
            pl.BlockSpec((1, H), lambda i: (0, 0)),
        ],
        out_specs=pl.BlockSpec((BN, H), lambda i: (i, 0)),
        out_shape=jax.ShapeDtypeStruct((N, H), jnp.float32),
    )(h, x0n, ln_msg_w.reshape(1, H), ln_msg_b.reshape(1, H))


def _k_ec(xi_ref, xj_ref, wa_ref, wb_ref, b1_ref, w2_ref, b2_ref, o_ref):
    t = _mm(xi_ref[...], wa_ref[...]) + _mm(xj_ref[...], wb_ref[...]) + b1_ref[...]
    o_ref[...] = _mm(_gelu(t), w2_ref[...]) + b2_ref[...]


def ec_mlp(xi, xj, w_a, w_b, ec_b1, ec_w2, ec_b2):
    return pl.pallas_call(
        _k_ec,
        grid=(EB,),
        in_specs=[
            pl.BlockSpec((BE, H), lambda i: (i, 0)),
            pl.BlockSpec((BE, H), lambda i: (i, 0)),
            pl.BlockSpec((H, H), lambda i: (0, 0)),
            pl.BlockSpec((H, H), lambda i: (0, 0)),
            pl.BlockSpec((1, H), lambda i: (0, 0)),
            pl.BlockSpec((H, H), lambda i: (0, 0)),
            pl.BlockSpec((1, H), lambda i: (0, 0)),
        ],
        out_specs=pl.BlockSpec((BE, H), lambda i: (i, 0)),
        out_shape=jax.ShapeDtypeStruct((E, H), jnp.float32),
    )(xi, xj, w_a, w_b, ec_b1.reshape(1, H), ec_w2, ec_b2.reshape(1, H))


def _k_fin(xn_ref, ec_ref, b_ref, w_ref, bia_ref, pool_ref):
    ec = ec_ref[...]
    ec = jnp.where(jnp.isfinite(ec), ec, 0.0)
    xn2 = xn_ref[...] + _ln(_gelu(ec), w_ref[...], bia_ref[...])
    bb = b_ref[0, 0, :]
    onehot = (bb[:, None] == lax.broadcasted_iota(jnp.int32, (BN, G), 1)).astype(jnp.float32)
    ps = _mm(onehot.T, xn2)

    @pl.when(pl.program_id(0) == 0)
    def _():
        pool_ref[...] = jnp.zeros_like(pool_ref)

    pool_ref[...] += ps


def fin_pool(xn, ecm, batch3, ln_edge_w, ln_edge_b):
    return pl.pallas_call(
        _k_fin,
        grid=(NB,),
        in_specs=[
            pl.BlockSpec((BN, H), lambda i: (i, 0)),
            pl.BlockSpec((BN, H), lambda i: (i, 0)),
            pl.BlockSpec((1, 1, BN), lambda i: (i, 0, 0)),
            pl.BlockSpec((1, H), lambda i: (0, 0)),
            pl.BlockSpec((1, H), lambda i: (0, 0)),
        ],
        out_specs=pl.BlockSpec((G, H), lambda i: (0, 0)),
        out_shape=jax.ShapeDtypeStruct((G, H), jnp.float32),
    )(xn, ecm, batch3, ln_edge_w.reshape(1, H), ln_edge_b.reshape(1, H))


def _k_proj(s_ref, c_ref, w_ref, b_ref, o_ref):
    cnt = jnp.maximum(c_ref[0, :], 1.0)[:, None]
    o_ref[...] = _mm(s_ref[...] / cnt, w_ref[...]) + b_ref[...]


def pool_proj(pool_sums, cnt, w_proj, b_proj):
    return pl.pallas_call(
        _k_proj,
        grid=(1,),
        in_specs=[
            pl.BlockSpec((G, H), lambda i: (0, 0)),
            pl.BlockSpec((1, G), lambda i: (0, 0)),
            pl.BlockSpec((H, P), lambda i: (0, 0)),
            pl.BlockSpec((1, P), lambda i: (0, 0)),
        ],
        out_specs=pl.BlockSpec((G, P), lambda i: (0, 0)),
        out_shape=jax.ShapeDtypeStruct((G, P), jnp.float32),
    )(pool_sums, cnt, w_proj, b_proj.reshape(1, P))


def kernel(x, edge_index, batch, w_in, b_in, gn_weight, gn_bias, gn_mean_scale,
           ggc_w, gru_w_ih, gru_w_hh, gru_b_ih, gru_b_hh,
           ln_msg_w, ln_msg_b, ln_edge_w, ln_edge_b,
           ec_w1, ec_b1, ec_w2, ec_b2, w_proj, b_proj):
    src = edge_index[0]
    dst = edge_index[1]
    batch3 = batch.reshape(NB, 1, BN)

    x0 = in_proj(x, w_in, b_in)
    sums, sq, cnt = graph_stats(x0, batch3)
    x0n = gn_apply(x0, batch3, sums, sq, cnt, gn_mean_scale, gn_weight, gn_bias)

    w_ihT = gru_w_ih.T
    w_hhT = gru_w_hh.T
    zer = jnp.zeros((RPS, HH), jnp.float32)
    h = x0n
    for i in range(L):
        m0, m1 = node_mm2(h, ggc_w[i])
        a0, a1 = sc_scatter_add(m0, m1, src, dst, zer)
        h = gru_cell(h, a0, a1, w_ihT, w_hhT, gru_b_ih, gru_b_hh)

    xn = msg_ln(h, x0n, ln_msg_w, ln_msg_b)

    w_a = ec_w1[:H] - ec_w1[H:]
    w_b = ec_w1[H:]
    xi = xn[dst]
    xj = xn[src]
    me = ec_mlp(xi, xj, w_a, w_b, ec_b1, ec_w2, ec_b2)
    ecm = jax.ops.segment_max(me, dst, num_segments=N)

    pool_sums = fin_pool(xn, ecm, batch3, ln_edge_w, ln_edge_b)
    return pool_proj(pool_sums, cnt, w_proj, b_proj)


# speedup vs baseline: 2.8788x; 1.2395x over previous
import functools
import math

import jax
import jax.numpy as jnp
from jax import lax
from jax.experimental import pallas as pl
from jax.experimental.pallas import tpu as pltpu
from jax.experimental.pallas import tpu_sc as plsc

N = 10000
E = 160000
D = 128
H = 256
L = 3
P = 128
G = 64

BN = 1000
BE = 1000
NB = N // BN
EB = E // BE

HH = H // 2
NSUB = 16
CE = 200
EPS = E // NSUB
NCH = EPS // CE
NPAD = 10240
RPS = NPAD // NSUB
RLAST = N - 15 * RPS

_SQRT2 = math.sqrt(2.0)


def _gelu(v):
    return 0.5 * v * (1.0 + lax.erf(v / _SQRT2))


def _ln(v, w, b):
    mu = jnp.mean(v, axis=-1, keepdims=True)
    var = jnp.mean((v - mu) ** 2, axis=-1, keepdims=True)
    return (v - mu) * lax.rsqrt(var + 1e-5) * w + b


def _mm(a, b):
    return jnp.dot(a, b, preferred_element_type=jnp.float32)


def _k_inproj(x_ref, w_ref, b_ref, o_ref):
    o_ref[...] = _mm(x_ref[...], w_ref[...]) + b_ref[...]


def in_proj(x, w_in, b_in):
    return pl.pallas_call(
        _k_inproj,
        grid=(NB,),
        in_specs=[
            pl.BlockSpec((BN, D), lambda i: (i, 0)),
            pl.BlockSpec((D, H), lambda i: (0, 0)),
            pl.BlockSpec((1, H), lambda i: (0, 0)),
        ],
        out_specs=pl.BlockSpec((BN, H), lambda i: (i, 0)),
        out_shape=jax.ShapeDtypeStruct((N, H), jnp.float32),
    )(x, w_in, b_in.reshape(1, H))


def _k_stats(x_ref, b_ref, sum_ref, sq_ref, cnt_ref):
    xb = x_ref[...]
    bb = b_ref[0, 0, :]
    onehot = (bb[:, None] == lax.broadcasted_iota(jnp.int32, (BN, G), 1)).astype(jnp.float32)
    ps = _mm(onehot.T, xb)
    pq = _mm(onehot.T, xb * xb)
    pc = jnp.sum(onehot, axis=0, keepdims=True)

    @pl.when(pl.program_id(0) == 0)
    def _():
        sum_ref[...] = jnp.zeros_like(sum_ref)
        sq_ref[...] = jnp.zeros_like(sq_ref)
        cnt_ref[...] = jnp.zeros_like(cnt_ref)

    sum_ref[...] += ps
    sq_ref[...] += pq
    cnt_ref[...] += pc


def graph_stats(x0, batch3):
    return pl.pallas_call(
        _k_stats,
        grid=(NB,),
        in_specs=[
            pl.BlockSpec((BN, H), lambda i: (i, 0)),
            pl.BlockSpec((1, 1, BN), lambda i: (i, 0, 0)),
        ],
        out_specs=[
            pl.BlockSpec((G, H), lambda i: (0, 0)),
            pl.BlockSpec((G, H), lambda i: (0, 0)),
            pl.BlockSpec((1, G), lambda i: (0, 0)),
        ],
        out_shape=[
            jax.ShapeDtypeStruct((G, H), jnp.float32),
            jax.ShapeDtypeStruct((G, H), jnp.float32),
            jax.ShapeDtypeStruct((1, G), jnp.float32),
        ],
    )(x0, batch3)


def _k_gn(x_ref, b_ref, sum_ref, sq_ref, cnt_ref, ms_ref, w_ref, bia_ref, o_ref):
    xb = x_ref[...]
    bb = b_ref[0, 0, :]
    cnt = jnp.maximum(cnt_ref[0, :], 1.0)[:, None]
    mean = sum_ref[...] / cnt
    s = ms_ref[...]
    m2 = mean * mean
    var = sq_ref[...] / cnt - 2.0 * s * m2 + (s * s) * m2
    onehot = (bb[:, None] == lax.broadcasted_iota(jnp.int32, (BN, G), 1)).astype(jnp.float32)
    mean_b = _mm(onehot, mean * s)
    var_b = _mm(onehot, var)
    o_ref[...] = (xb - mean_b) * lax.rsqrt(var_b + 1e-5) * w_ref[...] + bia_ref[...]


def gn_apply(x0, batch3, sums, sq, cnt, gn_mean_scale, gn_weight, gn_bias):
    return pl.pallas_call(
        _k_gn,
        grid=(NB,),
        in_specs=[
            pl.BlockSpec((BN, H), lambda i: (i, 0)),
            pl.BlockSpec((1, 1, BN), lambda i: (i, 0, 0)),
            pl.BlockSpec((G, H), lambda i: (0, 0)),
            pl.BlockSpec((G, H), lambda i: (0, 0)),
            pl.BlockSpec((1, G), lambda i: (0, 0)),
            pl.BlockSpec((1, H), lambda i: (0, 0)),
            pl.BlockSpec((1, H), lambda i: (0, 0)),
            pl.BlockSpec((1, H), lambda i: (0, 0)),
        ],
        out_specs=pl.BlockSpec((BN, H), lambda i: (i, 0)),
        out_shape=jax.ShapeDtypeStruct((N, H), jnp.float32),
    )(x0, batch3, sums, sq, cnt, gn_mean_scale.reshape(1, H),
      gn_weight.reshape(1, H), gn_bias.reshape(1, H))


def _k_mm2(a_ref, w_ref, o0_ref, o1_ref):
    r = _mm(a_ref[...], w_ref[...])
    o0_ref[...] = r[:, :HH]
    o1_ref[...] = r[:, HH:]


def node_mm2(h, w):
    return pl.pallas_call(
        _k_mm2,
        grid=(NB,),
        in_specs=[
            pl.BlockSpec((BN, H), lambda i: (i, 0)),
            pl.BlockSpec((H, H), lambda i: (0, 0)),
        ],
        out_specs=[
            pl.BlockSpec((BN, HH), lambda i: (i, 0)),
            pl.BlockSpec((BN, HH), lambda i: (i, 0)),
        ],
        out_shape=[
            jax.ShapeDtypeStruct((N, HH), jnp.float32),
            jax.ShapeDtypeStruct((N, HH), jnp.float32),
        ],
    )(h, w)


def _k_mm(a_ref, w_ref, o_ref):
    o_ref[...] = _mm(a_ref[...], w_ref[...])


def node_mm(h, w):
    return pl.pallas_call(
        _k_mm,
        grid=(NB,),
        in_specs=[
            pl.BlockSpec((BN, H), lambda i: (i, 0)),
            pl.BlockSpec((H, H), lambda i: (0, 0)),
        ],
        out_specs=pl.BlockSpec((BN, H), lambda i: (i, 0)),
        out_shape=jax.ShapeDtypeStruct((N, H), jnp.float32),
    )(h, w)


CG = 200
EPW = E // 32
NCHG = EPW // CG


def _sc_gather_body(p_hbm, q_hbm, src, dst, pd, qs, idx, rows, sem):
    cid = lax.axis_index("c")
    sid = lax.axis_index("s")
    wid = sid * 2 + cid

    def body(i, carry):
        base = wid * EPW + i * CG
        pltpu.sync_copy(dst.at[pl.ds(base, CG)], idx)
        pltpu.async_copy(p_hbm.at[idx], rows, sem).wait()
        pltpu.sync_copy(rows, pd.at[pl.ds(base, CG)])
        pltpu.sync_copy(src.at[pl.ds(base, CG)], idx)
        pltpu.async_copy(q_hbm.at[idx], rows, sem).wait()
        pltpu.sync_copy(rows, qs.at[pl.ds(base, CG)])
        return carry

    lax.fori_loop(0, NCHG, body, 0)


def sc_edge_gather(p, q, src, dst):
    mesh = plsc.VectorSubcoreMesh(core_axis_name="c", subcore_axis_name="s")
    f = functools.partial(
        pl.kernel,
        mesh=mesh,
        out_type=[
            jax.ShapeDtypeStruct((E, H), jnp.float32),
            jax.ShapeDtypeStruct((E, H), jnp.float32),
        ],
        scratch_types=[
            pltpu.VMEM((CG,), jnp.int32),
            pltpu.VMEM((CG, H), jnp.float32),
            pltpu.SemaphoreType.DMA,
        ],
    )(_sc_gather_body)
    return f(p, q, src, dst)


def _sc_scatter_body(m0, m1, src, dst, zer, o0, o1, acc, idx_s, idx_d, rows, sem):
    cid = lax.axis_index("c")
    sid = lax.axis_index("s")
    pltpu.sync_copy(zer, acc.at[pl.ds(sid * RPS, RPS)])
    plsc.subcore_barrier()

    def body(i, carry):
        base = sid * EPS + i * CE
        pltpu.sync_copy(src.at[pl.ds(base, CE)], idx_s)
        pltpu.sync_copy(dst.at[pl.ds(base, CE)], idx_d)

        @pl.when(cid == 0)
        def _():
            pltpu.async_copy(m0.at[idx_s], rows, sem).wait()

        @pl.when(cid == 1)
        def _():
            pltpu.async_copy(m1.at[idx_s], rows, sem).wait()

        pltpu.sync_copy(rows, acc.at[idx_d], add=True)
        return carry

    lax.fori_loop(0, NCH, body, 0)
    plsc.subcore_barrier()

    @pl.when(jnp.logical_and(cid == 0, sid < 15))
    def _():
        pltpu.sync_copy(acc.at[pl.ds(sid * RPS, RPS)], o0.at[pl.ds(sid * RPS, RPS)])

    @pl.when(jnp.logical_and(cid == 1, sid < 15))
    def _():
        pltpu.sync_copy(acc.at[pl.ds(sid * RPS, RPS)], o1.at[pl.ds(sid * RPS, RPS)])

    @pl.when(jnp.logical_and(cid == 0, sid == 15))
    def _():
        pltpu.sync_copy(acc.at[pl.ds(15 * RPS, RLAST)], o0.at[pl.ds(15 * RPS, RLAST)])

    @pl.when(jnp.logical_and(cid == 1, sid == 15))
    def _():
        pltpu.sync_copy(acc.at[pl.ds(15 * RPS, RLAST)], o1.at[pl.ds(15 * RPS, RLAST)])


def sc_scatter_add(m0, m1, src, dst, zer):
    mesh = plsc.VectorSubcoreMesh(core_axis_name="c", subcore_axis_name="s")
    f = functools.partial(
        pl.kernel,
        mesh=mesh,
        out_type=[
            jax.ShapeDtypeStruct((N, HH), jnp.float32),
            jax.ShapeDtypeStruct((N, HH), jnp.float32),
        ],
        scratch_types=[
            pltpu.VMEM_SHARED((NPAD, HH), jnp.float32),
            pltpu.VMEM((CE,), jnp.int32),
            pltpu.VMEM((CE,), jnp.int32),
            pltpu.VMEM((CE, HH), jnp.float32),
            pltpu.SemaphoreType.DMA,
        ],
    )(_sc_scatter_body)
    return f(m0, m1, src, dst, zer)


def _k_gru(h_ref, a0_ref, a1_ref, wih_ref, whh_ref, bih_ref, bhh_ref, o_ref):
    h = h_ref[...]
    wih = wih_ref[...]
    gi = _mm(a0_ref[...], wih[:HH, :]) + _mm(a1_ref[...], wih[HH:, :]) + bih_ref[...]
    gh = _mm(h, whh_ref[...]) + bhh_ref[...]
    i_r, i_z, i_n = gi[:, :H], gi[:, H:2 * H], gi[:, 2 * H:]
    h_r, h_z, h_n = gh[:, :H], gh[:, H:2 * H], gh[:, 2 * H:]
    r = jax.nn.sigmoid(i_r + h_r)
    z = jax.nn.sigmoid(i_z + h_z)
    n = jnp.tanh(i_n + r * h_n)
    o_ref[...] = (1.0 - z) * n + z * h


def gru_cell(h, a0, a1, w_ihT, w_hhT, b_ih, b_hh):
    return pl.pallas_call(
        _k_gru,
        grid=(NB,),
        in_specs=[
            pl.BlockSpec((BN, H), lambda i: (i, 0)),
            pl.BlockSpec((BN, HH), lambda i: (i, 0)),
            pl.BlockSpec((BN, HH), lambda i: (i, 0)),
            pl.BlockSpec((H, 3 * H), lambda i: (0, 0)),
            pl.BlockSpec((H, 3 * H), lambda i: (0, 0)),
            pl.BlockSpec((1, 3 * H), lambda i: (0, 0)),
            pl.BlockSpec((1, 3 * H), lambda i: (0, 0)),
        ],
        out_specs=pl.BlockSpec((BN, H), lambda i: (i, 0)),
        out_shape=jax.ShapeDtypeStruct((N, H), jnp.float32),
    )(h, a0, a1, w_ihT, w_hhT, b_ih.reshape(1, 3 * H), b_hh.reshape(1, 3 * H))


def _k_msgln(h_ref, x0_ref, w_ref, b_ref, o_ref):
    o_ref[...] = x0_ref[...] + _ln(_gelu(h_ref[...]), w_ref[...], b_ref[...])


def msg_ln(h, x0n, ln_msg_w, ln_msg_b):
    return pl.pallas_call(
        _k_msgln,
        grid=(NB,),
        in_specs=[
            pl.BlockSpec((BN, H), lambda i: (i, 0)),
            pl.BlockSpec((BN, H), lambda i: (i, 0)),
            pl.BlockSpec((1, H), lambda i: (0, 0)),
            pl.BlockSpec((1, H), lambda i: (0, 0)),
        ],
        out_specs=pl.BlockSpec((BN, H), lambda i: (i, 0)),
        out_shape=jax.ShapeDtypeStruct((N, H), jnp.float32),
    )(h, x0n, ln_msg_w.reshape(1, H), ln_msg_b.reshape(1, H))


def _k_ec(pd_ref, qs_ref, b1_ref, w2_ref, b2_ref, o_ref):
    t = pd_ref[...] + qs_ref[...] + b1_ref[...]
    o_ref[...] = _mm(_gelu(t), w2_ref[...]) + b2_ref[...]


def ec_mlp(pd, qs, ec_b1, ec_w2, ec_b2):
    return pl.pallas_call(
        _k_ec,
        grid=(EB,),
        in_specs=[
            pl.BlockSpec((BE, H), lambda i: (i, 0)),
            pl.BlockSpec((BE, H), lambda i: (i, 0)),
            pl.BlockSpec((1, H), lambda i: (0, 0)),
            pl.BlockSpec((H, H), lambda i: (0, 0)),
            pl.BlockSpec((1, H), lambda i: (0, 0)),
        ],
        out_specs=pl.BlockSpec((BE, H), lambda i: (i, 0)),
        out_shape=jax.ShapeDtypeStruct((E, H), jnp.float32),
    )(pd, qs, ec_b1.reshape(1, H), ec_w2, ec_b2.reshape(1, H))


def _k_fin(xn_ref, ec_ref, b_ref, w_ref, bia_ref, pool_ref):
    ec = ec_ref[...]
    ec = jnp.where(jnp.isfinite(ec), ec, 0.0)
    xn2 = xn_ref[...] + _ln(_gelu(ec), w_ref[...], bia_ref[...])
    bb = b_ref[0, 0, :]
    onehot = (bb[:, None] == lax.broadcasted_iota(jnp.int32, (BN, G), 1)).astype(jnp.float32)
    ps = _mm(onehot.T, xn2)

    @pl.when(pl.program_id(0) == 0)
    def _():
        pool_ref[...] = jnp.zeros_like(pool_ref)

    pool_ref[...] += ps


def fin_pool(xn, ecm, batch3, ln_edge_w, ln_edge_b):
    return pl.pallas_call(
        _k_fin,
        grid=(NB,),
        in_specs=[
            pl.BlockSpec((BN, H), lambda i: (i, 0)),
            pl.BlockSpec((BN, H), lambda i: (i, 0)),
            pl.BlockSpec((1, 1, BN), lambda i: (i, 0, 0)),
            pl.BlockSpec((1, H), lambda i: (0, 0)),
            pl.BlockSpec((1, H), lambda i: (0, 0)),
        ],
        out_specs=pl.BlockSpec((G, H), lambda i: (0, 0)),
        out_shape=jax.ShapeDtypeStruct((G, H), jnp.float32),
    )(xn, ecm, batch3, ln_edge_w.reshape(1, H), ln_edge_b.reshape(1, H))


def _k_proj(s_ref, c_ref, w_ref, b_ref, o_ref):
    cnt = jnp.maximum(c_ref[0, :], 1.0)[:, None]
    o_ref[...] = _mm(s_ref[...] / cnt, w_ref[...]) + b_ref[...]


def pool_proj(pool_sums, cnt, w_proj, b_proj):
    return pl.pallas_call(
        _k_proj,
        grid=(1,),
        in_specs=[
            pl.BlockSpec((G, H), lambda i: (0, 0)),
            pl.BlockSpec((1, G), lambda i: (0, 0)),
            pl.BlockSpec((H, P), lambda i: (0, 0)),
            pl.BlockSpec((1, P), lambda i: (0, 0)),
        ],
        out_specs=pl.BlockSpec((G, P), lambda i: (0, 0)),
        out_shape=jax.ShapeDtypeStruct((G, P), jnp.float32),
    )(pool_sums, cnt, w_proj, b_proj.reshape(1, P))


def kernel(x, edge_index, batch, w_in, b_in, gn_weight, gn_bias, gn_mean_scale,
           ggc_w, gru_w_ih, gru_w_hh, gru_b_ih, gru_b_hh,
           ln_msg_w, ln_msg_b, ln_edge_w, ln_edge_b,
           ec_w1, ec_b1, ec_w2, ec_b2, w_proj, b_proj):
    src = edge_index[0]
    dst = edge_index[1]
    batch3 = batch.reshape(NB, 1, BN)

    x0 = in_proj(x, w_in, b_in)
    sums, sq, cnt = graph_stats(x0, batch3)
    x0n = gn_apply(x0, batch3, sums, sq, cnt, gn_mean_scale, gn_weight, gn_bias)

    w_ihT = gru_w_ih.T
    w_hhT = gru_w_hh.T
    zer = jnp.zeros((RPS, HH), jnp.float32)
    h = x0n
    for i in range(L):
        m0, m1 = node_mm2(h, ggc_w[i])
        a0, a1 = sc_scatter_add(m0, m1, src, dst, zer)
        h = gru_cell(h, a0, a1, w_ihT, w_hhT, gru_b_ih, gru_b_hh)

    xn = msg_ln(h, x0n, ln_msg_w, ln_msg_b)

    w_a = ec_w1[:H] - ec_w1[H:]
    w_b = ec_w1[H:]
    p = node_mm(xn, w_a)
    q = node_mm(xn, w_b)
    pd, qs = sc_edge_gather(p, q, src, dst)
    me = ec_mlp(pd, qs, ec_b1, ec_w2, ec_b2)
    ecm = jax.ops.segment_max(me, dst, num_segments=N)

    pool_sums = fin_pool(xn, ecm, batch3, ln_edge_w, ln_edge_b)
    return pool_proj(pool_sums, cnt, w_proj, b_proj)
